# Initial kernel scaffold; baseline (speedup 1.0000x reference)
import jax
import jax.numpy as jnp
from jax.experimental import pallas as pl


def kernel(h, x, edge_attr, coarsening_matrix, t_emb, params, edge_index, cluster_assignment, edge_index_coarse):
    raise NotImplementedError("write your pallas kernel here")


# speedup vs baseline: 14.1449x; 14.1449x over previous
import functools

import jax
import jax.numpy as jnp
from jax import lax
from jax.experimental import pallas as pl
from jax.experimental.pallas import tpu as pltpu
from jax.experimental.pallas import tpu_sc as plsc

F32 = jnp.float32
SC_CORES = 2
SC_SUBCORES = 16
NWORK = SC_CORES * SC_SUBCORES
XW = 8


def _sc_mesh():
    return plsc.VectorSubcoreMesh(
        core_axis_name="c", subcore_axis_name="s",
        num_cores=SC_CORES, num_subcores=SC_SUBCORES)


@functools.lru_cache(maxsize=None)
def _make_gather(V, D, B, chunk):
    bpw = B // NWORK
    assert bpw * NWORK == B and bpw % chunk == 0 and chunk % 8 == 0
    nch = bpw // chunk

    def body(table, idx, out, idx_v, buf, sem):
        wid = lax.axis_index("s") * SC_CORES + lax.axis_index("c")
        base = pl.multiple_of(wid * bpw, 8)
        pltpu.sync_copy(idx.at[pl.ds(base, bpw)], idx_v)

        def step(j, carry):
            off = pl.multiple_of(base + j * chunk, 8)
            iv = idx_v.at[pl.ds(j * chunk, chunk)]
            pltpu.async_copy(table.at[iv], buf, sem).wait()
            pltpu.sync_copy(buf, out.at[pl.ds(off, chunk)])
            return carry

        lax.fori_loop(0, nch, step, 0)

    return pl.kernel(
        body,
        out_type=jax.ShapeDtypeStruct((B, D), F32),
        mesh=_sc_mesh(),
        compiler_params=pltpu.CompilerParams(use_tc_tiling_on_sc=False),
        scratch_types=[
            pltpu.VMEM((bpw,), jnp.int32),
            pltpu.VMEM((chunk, D), F32),
            pltpu.SemaphoreType.DMA,
        ],
    )


@functools.lru_cache(maxsize=None)
def _make_scatter(Ds, NSEG, B, chunk):
    bpw = B // NWORK
    assert bpw * NWORK == B and bpw % chunk == 0 and chunk % 8 == 0
    assert NSEG % (2 * SC_SUBCORES) == 0
    nch = bpw // chunk
    rows_p = NSEG // SC_SUBCORES
    nv = len(Ds)

    def body(*refs):
        vals = refs[:nv]
        idx = refs[nv]
        zeros = refs[nv + 1:2 * nv + 1]
        outs = refs[2 * nv + 1:3 * nv + 1]
        idx_v = refs[3 * nv + 1]
        val_vs = refs[3 * nv + 2:4 * nv + 2]
        accums = refs[4 * nv + 2:5 * nv + 2]

        cid = lax.axis_index("c")
        sid = lax.axis_index("s")
        wid = sid * SC_CORES + cid
        r0 = pl.multiple_of(sid * rows_p, 8)
        for z, acc in zip(zeros, accums):
            pltpu.sync_copy(z.at[pl.ds(r0, rows_p)], acc.at[pl.ds(r0, rows_p)])
        plsc.subcore_barrier()

        base = pl.multiple_of(wid * bpw, 8)

        def step(k, carry):
            off = pl.multiple_of(base + k * chunk, 8)
            pltpu.sync_copy(idx.at[pl.ds(off, chunk)], idx_v.at[k])
            for vi in range(nv):
                pltpu.sync_copy(vals[vi].at[pl.ds(off, chunk)], val_vs[vi])
                pltpu.sync_copy(val_vs[vi], accums[vi].at[idx_v.at[k]],
                                add=True)
            return carry

        lax.fori_loop(0, nch, step, 0)
        plsc.subcore_barrier()
        for acc, out in zip(accums, outs):
            pltpu.sync_copy(acc.at[pl.ds(r0, rows_p)],
                            out.at[cid, pl.ds(r0, rows_p)])

    scratch = [pltpu.VMEM((nch, chunk), jnp.int32)]
    scratch += [pltpu.VMEM((chunk, D), F32) for D in Ds]
    scratch += [pltpu.VMEM_SHARED((NSEG, D), F32) for D in Ds]

    return pl.kernel(
        body,
        out_type=[jax.ShapeDtypeStruct((SC_CORES, NSEG, D), F32) for D in Ds],
        mesh=_sc_mesh(),
        compiler_params=pltpu.CompilerParams(use_tc_tiling_on_sc=False),
        scratch_types=scratch,
    )


def _scatter(vals, idx, nseg, chunk):
    Ds = tuple(int(v.shape[1]) for v in vals)
    B = int(idx.shape[0])
    fn = _make_scatter(Ds, nseg, B, chunk)
    zeros = [jnp.zeros((nseg, D), F32) for D in Ds]
    return fn(*vals, idx, *zeros)


def _silu(v):
    return v * jax.nn.sigmoid(v)


@functools.lru_cache(maxsize=None)
def _make_edge(D, E_pad, E_real, T, has_ea):
    grid = E_pad // T

    def kern(*refs):
        if has_ea:
            gA, gB, xs, xd, ea8, We2, Wx1, Wea, aux, m_out, t_out = refs
        else:
            gA, gB, xs, xd, We2, Wx1, aux, m_out, t_out = refs
        diff = xs[...] - xd[...]
        dist2 = jnp.sum(diff * diff, axis=1, keepdims=True)
        m1 = gA[...] + gB[...] + dist2 * aux[3:4, :] + aux[0:1, :]
        if has_ea:
            m1 = m1 + jnp.dot(ea8[...], Wea[...], preferred_element_type=F32)
        m1 = _silu(m1)
        m2 = jnp.dot(m1, We2[...], preferred_element_type=F32) + aux[1:2, :]
        m2 = _silu(m2)
        att = jax.nn.sigmoid(
            jnp.sum(m2 * aux[4:5, :], axis=1, keepdims=True) + aux[6:7, 0:1])
        m = m2 * att
        p1 = jnp.dot(m, Wx1[...], preferred_element_type=F32) + aux[2:3, :]
        p1 = _silu(p1)
        phi = jnp.sum(p1 * aux[5:6, :], axis=1, keepdims=True) + aux[7:8, 0:1]
        tr = diff * phi
        if E_real != E_pad:
            rid = (pl.program_id(0) * T
                   + lax.broadcasted_iota(jnp.int32, (T, 1), 0))
            valid = rid < E_real
            m = jnp.where(valid, m, 0.0)
            tr = jnp.where(valid, tr, 0.0)
        m_out[...] = m
        t_out[...] = tr

    tile = lambda w: pl.BlockSpec((T, w), lambda i: (i, 0))
    full = lambda a, b: pl.BlockSpec((a, b), lambda i: (0, 0))
    if has_ea:
        in_specs = [tile(D), tile(D), tile(XW), tile(XW), tile(8),
                    full(D, D), full(D, D), full(8, D), full(8, D)]
    else:
        in_specs = [tile(D), tile(D), tile(XW), tile(XW),
                    full(D, D), full(D, D), full(8, D)]
    return pl.pallas_call(
        kern,
        grid=(grid,),
        in_specs=in_specs,
        out_specs=[pl.BlockSpec((T, D), lambda i: (i, 0)),
                   pl.BlockSpec((T, XW), lambda i: (i, 0))],
        out_shape=[jax.ShapeDtypeStruct((E_pad, D), F32),
                   jax.ShapeDtypeStruct((E_pad, XW), F32)],
    )


@functools.lru_cache(maxsize=None)
def _make_node(D, TDW, NROW, T):
    grid = NROW // T

    def kern(h, t, am0, am1, ax0, ax1, cn0, cn1, x8,
             W1h, W1m, W1t, Wn2, aux, h_out, x_out):
        cnt = (cn0[...] + cn1[...])[:, 0:1]
        inv = 1.0 / jnp.maximum(cnt, 1.0)
        x_out[...] = x8[...] + (ax0[...] + ax1[...]) * inv
        am = am0[...] + am1[...]
        u = (jnp.dot(h[...], W1h[...], preferred_element_type=F32)
             + jnp.dot(am, W1m[...], preferred_element_type=F32)
             + jnp.dot(t[...], W1t[...], preferred_element_type=F32)
             + aux[0:1, :])
        u = _silu(u)
        h_out[...] = h[...] + jnp.dot(u, Wn2[...],
                                      preferred_element_type=F32) + aux[1:2, :]

    tile = lambda w: pl.BlockSpec((T, w), lambda i: (i, 0))
    full = lambda a, b: pl.BlockSpec((a, b), lambda i: (0, 0))
    return pl.pallas_call(
        kern,
        grid=(grid,),
        in_specs=[tile(D), tile(TDW), tile(D), tile(D), tile(XW), tile(XW),
                  tile(XW), tile(XW), tile(XW),
                  full(D, D), full(D, D), full(TDW, D), full(D, D),
                  full(8, D)],
        out_specs=[pl.BlockSpec((T, D), lambda i: (i, 0)),
                   pl.BlockSpec((T, XW), lambda i: (i, 0))],
        out_shape=[jax.ShapeDtypeStruct((NROW, D), F32),
                   jax.ShapeDtypeStruct((NROW, XW), F32)],
    )


@functools.lru_cache(maxsize=None)
def _make_dual_mm(K, D, NROW, T):
    grid = NROW // T

    def kern(X, W1, W2, o1, o2):
        o1[...] = jnp.dot(X[...], W1[...], preferred_element_type=F32)
        o2[...] = jnp.dot(X[...], W2[...], preferred_element_type=F32)

    return pl.pallas_call(
        kern,
        grid=(grid,),
        in_specs=[pl.BlockSpec((T, K), lambda i: (i, 0)),
                  pl.BlockSpec((K, D), lambda i: (0, 0)),
                  pl.BlockSpec((K, D), lambda i: (0, 0))],
        out_specs=[pl.BlockSpec((T, D), lambda i: (i, 0)),
                   pl.BlockSpec((T, D), lambda i: (i, 0))],
        out_shape=[jax.ShapeDtypeStruct((NROW, D), F32),
                   jax.ShapeDtypeStruct((NROW, D), F32)],
    )


@functools.lru_cache(maxsize=None)
def _make_gram(NCr, NTOT, TK):
    assert NTOT % TK == 0 and TK % 128 == 0
    grid = NTOT // TK

    def kern(Cb, Ctb, x8b, G, xc):
        @pl.when(pl.program_id(0) == 0)
        def _():
            G[...] = jnp.zeros_like(G)
            xc[...] = jnp.zeros_like(xc)

        G[...] += jnp.dot(Cb[...], Ctb[...], preferred_element_type=F32)
        xc[...] += jnp.dot(Cb[...], x8b[...], preferred_element_type=F32)

    return pl.pallas_call(
        kern,
        grid=(grid,),
        in_specs=[pl.BlockSpec((NCr, TK), lambda i: (0, i)),
                  pl.BlockSpec((TK, NCr), lambda i: (i, 0)),
                  pl.BlockSpec((TK, XW), lambda i: (i, 0))],
        out_specs=[pl.BlockSpec((NCr, NCr), lambda i: (0, 0)),
                   pl.BlockSpec((NCr, XW), lambda i: (0, 0))],
        out_shape=[jax.ShapeDtypeStruct((NCr, NCr), F32),
                   jax.ShapeDtypeStruct((NCr, XW), F32)],
    )


@functools.lru_cache(maxsize=None)
def _make_cg(NCr, iters):
    def kern(G_ref, b_ref, x_ref):
        G = G_ref[...]
        b = b_ref[...]

        def body(i, st):
            X, R, P, rs = st
            Q = jnp.dot(G, P, preferred_element_type=F32,
                        precision=lax.Precision.HIGHEST)
            den = jnp.sum(P * Q, axis=0, keepdims=True)
            alpha = jnp.where(den > 0, rs / den, 0.0)
            X = X + alpha * P
            R = R - alpha * Q
            rs2 = jnp.sum(R * R, axis=0, keepdims=True)
            beta = jnp.where(rs > 0, rs2 / rs, 0.0)
            P = R + beta * P
            return (X, R, P, rs2)

        X0 = jnp.zeros_like(b)
        rs0 = jnp.sum(b * b, axis=0, keepdims=True)
        X, _, _, _ = lax.fori_loop(0, iters, body, (X0, b, b, rs0))
        x_ref[...] = X

    return pl.pallas_call(
        kern,
        out_shape=jax.ShapeDtypeStruct((NCr, XW), F32),
    )


@functools.lru_cache(maxsize=None)
def _make_xf(NCr, NTOT, TN):
    grid = NTOT // TN

    def kern(Ctb, Xf, out):
        out[...] = jnp.dot(Ctb[...], Xf[...], preferred_element_type=F32,
                           precision=lax.Precision.HIGHEST)

    return pl.pallas_call(
        kern,
        grid=(grid,),
        in_specs=[pl.BlockSpec((TN, NCr), lambda i: (i, 0)),
                  pl.BlockSpec((NCr, XW), lambda i: (0, 0))],
        out_specs=pl.BlockSpec((TN, XW), lambda i: (i, 0)),
        out_shape=jax.ShapeDtypeStruct((NTOT, XW), F32),
    )


@functools.lru_cache(maxsize=None)
def _make_poolfin(NCr, D0, D1):
    def kern(ah0, ah1, at0, at1, cn0, cn1, Wp, aux, hc_out, tc_out):
        cnt = (cn0[...] + cn1[...])[:, 0:1]
        inv = 1.0 / jnp.maximum(cnt, 1.0)
        hm = (ah0[...] + ah1[...]) * inv
        tm = (at0[...] + at1[...]) * inv
        hc_out[...] = jnp.dot(hm, Wp[...],
                              preferred_element_type=F32) + aux[0:1, :]
        tc_out[...] = tm

    return pl.pallas_call(
        kern,
        out_shape=[jax.ShapeDtypeStruct((NCr, D1), F32),
                   jax.ShapeDtypeStruct((NCr, D0), F32)],
    )


@functools.lru_cache(maxsize=None)
def _make_mm(M, K, D):
    def kern(X, W, o):
        o[...] = jnp.dot(X[...], W[...], preferred_element_type=F32)

    return pl.pallas_call(
        kern, out_shape=jax.ShapeDtypeStruct((M, D), F32))


@functools.lru_cache(maxsize=None)
def _make_skip(NROW, D0, T):
    grid = NROW // T

    def kern(he, hf, Wsk, aux, g_row, out):
        g = g_row[0:1, :]
        sk = jnp.dot(he[...], Wsk[...], preferred_element_type=F32) + aux[0:1, :]
        out[...] = g * sk + (1.0 - g) * (hf[...] + aux[1:2, :])

    tile = lambda w: pl.BlockSpec((T, w), lambda i: (i, 0))
    full = lambda a, b: pl.BlockSpec((a, b), lambda i: (0, 0))
    return pl.pallas_call(
        kern,
        grid=(grid,),
        in_specs=[tile(D0), tile(D0), full(D0, D0), full(8, D0), full(8, D0)],
        out_specs=pl.BlockSpec((T, D0), lambda i: (i, 0)),
        out_shape=jax.ShapeDtypeStruct((NROW, D0), F32),
    )


def _edge_aux(p, D):
    aux = jnp.zeros((8, D), F32)
    aux = aux.at[0].set(p['be1'])
    aux = aux.at[1].set(p['be2'])
    aux = aux.at[2].set(p['bx1'])
    aux = aux.at[3].set(p['We1'][2 * D])
    aux = aux.at[4].set(p['Wa'][:, 0])
    aux = aux.at[5].set(p['Wx2'][:, 0])
    aux = aux.at[6, 0].set(p['ba'][0])
    aux = aux.at[7, 0].set(p['bx2'][0])
    return aux


def _node_aux(p, D):
    aux = jnp.zeros((8, D), F32)
    aux = aux.at[0].set(p['bn1'])
    aux = aux.at[1].set(p['bn2'])
    return aux


def kernel(h, x, edge_attr, coarsening_matrix, t_emb, params, edge_index,
           cluster_assignment, edge_index_coarse):
    N, D0 = h.shape
    NCr = coarsening_matrix.shape[0]
    E = edge_index.shape[1]
    EC = edge_index_coarse.shape[1]
    TD = t_emb.shape[1]
    D1 = params['Wpool'].shape[1]

    EC_pad = 16384
    N_pad = 10240
    N_acc = 10240
    NC_acc = 1024

    C = coarsening_matrix
    Ct = jnp.swapaxes(C, 0, 1)

    src = edge_index[0]
    dst = edge_index[1]
    sc_pad = jnp.concatenate([edge_index_coarse[0],
                              jnp.zeros((EC_pad - EC,), jnp.int32)])
    dc_pad = jnp.concatenate([edge_index_coarse[1],
                              jnp.zeros((EC_pad - EC,), jnp.int32)])
    clu_pad = jnp.concatenate([cluster_assignment,
                               jnp.zeros((N_pad - N,), jnp.int32)])

    x8 = jnp.concatenate([x, jnp.zeros((N, XW - 3), F32)], axis=1)
    ea8 = jnp.concatenate([edge_attr,
                           jnp.zeros((E, 8 - edge_attr.shape[1]), F32)], axis=1)

    g_f128 = _make_gather(N, D0, E, 200)
    g_f8 = _make_gather(N, XW, E, 1000)
    g_c256 = _make_gather(NCr, D1, EC_pad, 128)
    g_c8 = _make_gather(NCr, XW, EC_pad, 512)
    g_un = _make_gather(NCr, D0, N_pad, 160)

    edge_f = _make_edge(D0, E, E, 2000, True)
    edge_c = _make_edge(D1, EC_pad, EC, 2048, False)
    node_f = _make_node(D0, TD, N, 2000)
    node_c = _make_node(D1, TD, NCr, NCr)
    pre_f = _make_dual_mm(D0, D0, N, 2000)
    pre_c = _make_dual_mm(D1, D1, NCr, NCr)

    ones_f = jnp.ones((E, 8), F32)
    ones_c = jnp.concatenate([jnp.ones((EC, 8), F32),
                              jnp.zeros((EC_pad - EC, 8), F32)])
    ones_n = jnp.concatenate([jnp.ones((N, 8), F32),
                              jnp.zeros((N_pad - N, 8), F32)])
    (cntf,) = _scatter((ones_f,), dst, N_acc, 1000)
    (cntc,) = _scatter((ones_c,), dc_pad, NC_acc, 512)

    def fine_block(p, hh, xx8):
        hA, hB = pre_f(hh, p['We1'][0:D0], p['We1'][D0:2 * D0])
        gA = g_f128(hA, src)
        gB = g_f128(hB, dst)
        gxs = g_f8(xx8, src)
        gxd = g_f8(xx8, dst)
        Wea = jnp.concatenate(
            [p['We1'][2 * D0 + 1:], jnp.zeros((4, D0), F32)], axis=0)
        m, tr = edge_f(gA, gB, gxs, gxd, ea8, p['We2'], p['Wx1'], Wea,
                       _edge_aux(p, D0))
        am, ax = _scatter((m, tr), dst, N_acc, 200)
        return node_f(hh, t_emb, am[0, :N], am[1, :N], ax[0, :N], ax[1, :N],
                      cntf[0, :N], cntf[1, :N], xx8,
                      p['Wn1'][0:D0], p['Wn1'][D0:2 * D0], p['Wn1'][2 * D0:],
                      p['Wn2'], _node_aux(p, D0))

    def coarse_block(p, hh, xx8, tt):
        hA, hB = pre_c(hh, p['We1'][0:D1], p['We1'][D1:2 * D1])
        gA = g_c256(hA, sc_pad)
        gB = g_c256(hB, dc_pad)
        gxs = g_c8(xx8, sc_pad)
        gxd = g_c8(xx8, dc_pad)
        m, tr = edge_c(gA, gB, gxs, gxd, p['We2'], p['Wx1'], _edge_aux(p, D1))
        am, ax = _scatter((m, tr), dc_pad, NC_acc, 256)
        return node_c(hh, tt, am[0, :NCr], am[1, :NCr],
                      ax[0, :NCr], ax[1, :NCr],
                      cntc[0, :NCr], cntc[1, :NCr], xx8,
                      p['Wn1'][0:D1], p['Wn1'][D1:2 * D1], p['Wn1'][2 * D1:],
                      p['Wn2'], _node_aux(p, D1))

    for p in params['enc0']:
        h, x8 = fine_block(p, h, x8)
    h_enc = h

    h_pad = jnp.concatenate([h, jnp.zeros((N_pad - N, D0), F32)])
    t_pad = jnp.concatenate([t_emb, jnp.zeros((N_pad - N, TD), F32)])
    aggh, aggt, ccnt = _scatter((h_pad, t_pad, ones_n), clu_pad, NC_acc, 320)
    C_pad = jnp.concatenate([C, jnp.zeros((NCr, N_pad - N), F32)], axis=1)
    Ct_pad = jnp.concatenate([Ct, jnp.zeros((N_pad - N, NCr), F32)], axis=0)
    x8_pad = jnp.concatenate([x8, jnp.zeros((N_pad - N, XW), F32)], axis=0)
    G, xc8 = _make_gram(NCr, N_pad, 2048)(C_pad, Ct_pad, x8_pad)
    h_c, t_c = _make_poolfin(NCr, D0, D1)(
        aggh[0, :NCr], aggh[1, :NCr], aggt[0, :NCr], aggt[1, :NCr],
        ccnt[0, :NCr], ccnt[1, :NCr], params['Wpool'],
        jnp.zeros((8, D1), F32).at[0].set(params['bpool']))

    for p in params['lvl1']:
        h_c, xc8 = coarse_block(p, h_c, xc8, t_c)

    Y = _make_cg(NCr, 48)(G, xc8)
    xf8 = _make_xf(NCr, N_pad, 2048)(Ct_pad, Y)[:N]
    hcW = _make_mm(NCr, D1, D0)(h_c, params['Wun'])
    h_f = g_un(hcW, clu_pad)[:N]

    skip_aux = (jnp.zeros((8, D0), F32)
                .at[0].set(params['bskip']).at[1].set(params['bun']))
    g_row = jnp.broadcast_to(params['gate'].reshape(1, 1), (8, D0)).astype(F32)
    h_d = _make_skip(N, D0, 2000)(h_enc, h_f, params['Wskip'], skip_aux, g_row)

    for p in params['dec0']:
        h_d, xf8 = fine_block(p, h_d, xf8)

    return h_d, xf8[:, :3]
